# baseline (speedup 1.0000x reference)
import functools
import math

import jax
import jax.numpy as jnp
import numpy as np
from jax import lax
from jax.experimental import pallas as pl
from jax.experimental.pallas import tpu as pltpu

_EPS = 1e-5
_N = 8
_H0 = 256
_H1 = 128
_K = 5

_BF = jnp.bfloat16


def _shiftr(xe, d, H):
    if d == 0:
        return xe
    z = jnp.zeros((abs(d), xe.shape[1]), xe.dtype)
    if d > 0:
        return jnp.concatenate([xe[d:, :], z], axis=0)
    return jnp.concatenate([z, xe[:H + d, :]], axis=0)


def _conv_same(x, w3, *, K, H, n):
    pad = K // 2
    xb = [x[e * H:(e + 1) * H, :].astype(_BF) for e in range(n)]
    acc = None
    for t in range(K):
        d = t - pad
        shf = jnp.concatenate([_shiftr(xe, d, H) for xe in xb], axis=0)
        dt = jnp.dot(shf, w3[t], preferred_element_type=jnp.float32)
        acc = dt if acc is None else acc + dt
    return acc


def _gn_mish(acc, sel, b, g, beta, gavg, *, H, n, post=None):
    cout = acc.shape[1]
    cc = jnp.concatenate([acc, jnp.square(acc)], axis=1)
    s = jnp.dot(sel, cc, preferred_element_type=jnp.float32)
    m_raw = s[:, :cout]
    q_raw = s[:, cout:]
    m = m_raw + b
    q = q_raw + 2.0 * b * m_raw + b * b
    gs = jnp.dot(jnp.concatenate([m, q], axis=0), gavg,
                 preferred_element_type=jnp.float32)
    gm = gs[:n, :]
    var = jnp.maximum(gs[n:, :] - gm * gm, 0.0)
    sc = g * lax.rsqrt(var + _EPS)
    sh = (b - gm) * sc + beta
    outs = []
    for e in range(n):
        y = acc[e * H:(e + 1) * H, :] * sc[e:e + 1, :] + sh[e:e + 1, :]
        t2 = jnp.square(1.0 + jnp.exp(jnp.minimum(y, 20.0)))
        o = y * (t2 - 1.0) / (t2 + 1.0)
        if post is not None:
            o = o + post[e:e + 1, :]
        outs.append(o)
    return jnp.concatenate(outs, axis=0)


def _rb(x, mtm, sel, w0, w1, tw, pvt, gavg, wr, *, K, H, n):
    pv = pvt[...]
    tb = jnp.dot(mtm, tw[...], preferred_element_type=jnp.float32) \
        + pv[7:8, :]
    a0 = _conv_same(x, w0, K=K, H=H, n=n)
    h = _gn_mish(a0, sel, pv[0:1, :], pv[1:2, :], pv[2:3, :], gavg,
                 H=H, n=n, post=tb)
    a1 = _conv_same(h, w1, K=K, H=H, n=n)
    y = _gn_mish(a1, sel, pv[3:4, :], pv[4:5, :], pv[5:6, :], gavg, H=H, n=n)
    if wr is None:
        res = x
    else:
        res = jnp.dot(x.astype(_BF), wr[...],
                      preferred_element_type=jnp.float32) + pv[6:7, :]
    return y + res


def _unet_kernel(x_ref, mt_ref, sel0_ref, sel1_ref,
                 a_w0, a_w1, a_tw, a_pv, a_wr,
                 b_w0, b_w1, b_tw, b_pv,
                 g256, g512,
                 dw3, dsel, ddb,
                 c_w0, c_w1, c_tw, c_pv, c_wr,
                 d_w0, d_w1, d_tw, d_pv,
                 e_w0, e_w1, e_tw, e_pv,
                 f_w0_, f_w1, f_tw, f_pv,
                 p_w0, p_w1, p_tw, p_pv, p_wr,
                 q_w0, q_w1, q_tw, q_pv,
                 uwe, uwo, upe, upo, uub,
                 z_w0, z_pv, z_wf, z_bf,
                 o_ref, *, n):
    H0, H1, K = _H0, _H1, _K
    ga = g256[...]
    gb = g512[...]
    sel0 = sel0_ref[...]
    sel1 = sel1_ref[...]

    x = jnp.concatenate([x_ref[e].astype(jnp.float32) for e in range(n)],
                        axis=0)
    mtm = jnp.concatenate([mt_ref[e] for e in range(n)],
                          axis=0).astype(_BF)

    x = _rb(x, mtm, sel0, a_w0, a_w1, a_tw, a_pv, ga, a_wr, K=K, H=H0, n=n)
    x = _rb(x, mtm, sel0, b_w0, b_w1, b_tw, b_pv, ga, None, K=K, H=H0, n=n)

    parts = []
    for e in range(n):
        xe = x[e * H0:(e + 1) * H0, :].astype(_BF)
        gath = jnp.dot(dsel[...], xe,
                       preferred_element_type=jnp.float32)
        gb16 = gath.astype(_BF)
        acc = None
        for t in range(3):
            dt = jnp.dot(gb16[t * H1:(t + 1) * H1, :], dw3[t],
                         preferred_element_type=jnp.float32)
            acc = dt if acc is None else acc + dt
        parts.append(acc)
    x = jnp.concatenate(parts, axis=0) + ddb[...]

    x = _rb(x, mtm, sel1, c_w0, c_w1, c_tw, c_pv, gb, c_wr, K=K, H=H1, n=n)
    x = _rb(x, mtm, sel1, d_w0, d_w1, d_tw, d_pv, gb, None, K=K, H=H1, n=n)
    skip = x

    x = _rb(x, mtm, sel1, e_w0, e_w1, e_tw, e_pv, gb, None, K=K, H=H1, n=n)
    x = _rb(x, mtm, sel1, f_w0_, f_w1, f_tw, f_pv, gb, None, K=K, H=H1, n=n)

    x = jnp.concatenate([x, skip], axis=1)
    x = _rb(x, mtm, sel1, p_w0, p_w1, p_tw, p_pv, ga, p_wr, K=K, H=H1, n=n)
    x = _rb(x, mtm, sel1, q_w0, q_w1, q_tw, q_pv, ga, None, K=K, H=H1, n=n)

    parts = []
    for e in range(n):
        xe = x[e * H1:(e + 1) * H1, :].astype(_BF)
        xm1 = _shiftr(xe, -1, H1)
        xp1 = _shiftr(xe, 1, H1)
        ev = jnp.dot(jnp.concatenate([xm1, xe], axis=1), uwe[...],
                     preferred_element_type=jnp.float32)
        od = jnp.dot(jnp.concatenate([xe, xp1], axis=1), uwo[...],
                     preferred_element_type=jnp.float32)
        parts.append(
            jnp.dot(upe[...], ev.astype(_BF),
                    preferred_element_type=jnp.float32)
            + jnp.dot(upo[...], od.astype(_BF),
                      preferred_element_type=jnp.float32))
    x = jnp.concatenate(parts, axis=0) + uub[...]

    pvz = z_pv[...]
    a = _conv_same(x, z_w0, K=K, H=H0, n=n)
    y = _gn_mish(a, sel0, pvz[0:1, :], pvz[1:2, :], pvz[2:3, :], ga,
                 H=H0, n=n)
    out = jnp.dot(y.astype(_BF), z_wf[...],
                  preferred_element_type=jnp.float32) + z_bf[...]
    for e in range(n):
        o_ref[e] = out[e * H0:(e + 1) * H0, :]


def _mish(v):
    return v * jnp.tanh(jax.nn.softplus(v))


def _sin_emb(t, dim):
    half = dim // 2
    freq = jnp.exp(jnp.arange(half, dtype=jnp.float32)
                   * (-math.log(10000.0) / (half - 1)))
    args = t.astype(jnp.float32)[:, None] * freq[None, :]
    return jnp.concatenate([jnp.sin(args), jnp.cos(args)], axis=-1)


def _cspec(shape):
    nd = len(shape)
    return pl.BlockSpec(shape, lambda i: (0,) * nd)


@functools.lru_cache(maxsize=None)
def _sel_matrices():
    s = np.zeros((3 * _H1, _H0), np.float32)
    for t in range(3):
        for j in range(_H1):
            m = 2 * j + t - 1
            if 0 <= m < _H0:
                s[t * _H1 + j, m] = 1.0
    pe = np.zeros((_H0, _H1), np.float32)
    po = np.zeros((_H0, _H1), np.float32)
    pe[2 * np.arange(_H1), np.arange(_H1)] = 1.0
    po[2 * np.arange(_H1) + 1, np.arange(_H1)] = 1.0
    s0 = np.zeros((_N, _N * _H0), np.float32)
    s1 = np.zeros((_N, _N * _H1), np.float32)
    for e in range(_N):
        s0[e, e * _H0:(e + 1) * _H0] = 1.0 / _H0
        s1[e, e * _H1:(e + 1) * _H1] = 1.0 / _H1
    return s, pe, po, s0, s1


def _tapw(w, cin):
    cout = w.shape[0]
    k = w.shape[1] // cin
    return jnp.transpose(w.reshape(cout, k, cin), (1, 2, 0)).astype(_BF)


def kernel(tm_w1, tm_b1, tm_w2, tm_b2, rm_w1, rm_b1, rm_w2, rm_b2, rm_w3,
           rm_b3, d0r1_w0, d0r1_w1, d0r1_tw, d0r1_pv, d0r1_gavg, d0r1_wr,
           d0r2_w0, d0r2_w1, d0r2_tw, d0r2_pv, d0r2_gavg, d0_dw, d0_db,
           d1r1_w0, d1r1_w1, d1r1_tw, d1r1_pv, d1r1_gavg, d1r1_wr,
           d1r2_w0, d1r2_w1, d1r2_tw, d1r2_pv, d1r2_gavg,
           m1_w0, m1_w1, m1_tw, m1_pv, m1_gavg,
           m2_w0, m2_w1, m2_tw, m2_pv, m2_gavg,
           u0r1_w0, u0r1_w1, u0r1_tw, u0r1_pv, u0r1_gavg, u0r1_wr,
           u0r2_w0, u0r2_w1, u0r2_tw, u0r2_pv, u0r2_gavg,
           u0_uw, u0_ub, f_w0, f_pv, f_gavg, f_wf, f_bf, x, time, returns):
    B = x.shape[0]
    n = _N
    dim = 256

    e = _sin_emb(time, dim)
    e = _mish(jnp.dot(e, tm_w1.T) + tm_b1)
    t = jnp.dot(e, tm_w2.T) + tm_b2
    r = _mish(jnp.dot(returns, rm_w1.T) + rm_b1)
    r = _mish(jnp.dot(r, rm_w2.T) + rm_b2)
    r = jnp.dot(r, rm_w3.T) + rm_b3
    mt = _mish(jnp.concatenate([t, r], axis=-1))
    mt = mt.reshape(B, 1, 512)

    bf = lambda a: a.astype(_BF)

    dw3 = bf(jnp.transpose(d0_dw, (2, 1, 0)))
    ddb = d0_db.reshape(1, -1)

    uwe = bf(jnp.concatenate([u0_uw[:, :, 0].T, u0_uw[:, :, 2].T], axis=0))
    uwo = bf(jnp.concatenate([u0_uw[:, :, 1].T, u0_uw[:, :, 3].T], axis=0))
    uub = u0_ub.reshape(1, -1)

    s_np, pe_np, po_np, s0_np, s1_np = _sel_matrices()
    dsel = jnp.asarray(s_np, _BF)
    upe = jnp.asarray(pe_np, _BF)
    upo = jnp.asarray(po_np, _BF)
    sel0 = jnp.asarray(s0_np)
    sel1 = jnp.asarray(s1_np)

    pvt = lambda p: p.T

    consts = [
        (_tapw(d0r1_w0, 32), _tapw(d0r1_w1, 256), bf(d0r1_tw.T),
         pvt(d0r1_pv), bf(d0r1_wr.T)),
        (_tapw(d0r2_w0, 256), _tapw(d0r2_w1, 256), bf(d0r2_tw.T),
         pvt(d0r2_pv)),
        (d0r1_gavg, d1r1_gavg),
        (dw3, dsel, ddb),
        (_tapw(d1r1_w0, 256), _tapw(d1r1_w1, 512), bf(d1r1_tw.T),
         pvt(d1r1_pv), bf(d1r1_wr.T)),
        (_tapw(d1r2_w0, 512), _tapw(d1r2_w1, 512), bf(d1r2_tw.T),
         pvt(d1r2_pv)),
        (_tapw(m1_w0, 512), _tapw(m1_w1, 512), bf(m1_tw.T), pvt(m1_pv)),
        (_tapw(m2_w0, 512), _tapw(m2_w1, 512), bf(m2_tw.T), pvt(m2_pv)),
        (_tapw(u0r1_w0, 1024), _tapw(u0r1_w1, 256), bf(u0r1_tw.T),
         pvt(u0r1_pv), bf(u0r1_wr.T)),
        (_tapw(u0r2_w0, 256), _tapw(u0r2_w1, 256), bf(u0r2_tw.T),
         pvt(u0r2_pv)),
        (uwe, uwo, upe, upo, uub),
        (_tapw(f_w0, 256), pvt(f_pv), bf(f_wf.T), f_bf.reshape(1, -1)),
    ]
    flat = [a for grp in consts for a in grp]

    in_specs = [
        pl.BlockSpec((n, _H0, 32), lambda i: (i, 0, 0)),
        pl.BlockSpec((n, 1, 512), lambda i: (i, 0, 0)),
        _cspec(sel0.shape),
        _cspec(sel1.shape),
    ] + [_cspec(a.shape) for a in flat]

    out = pl.pallas_call(
        functools.partial(_unet_kernel, n=n),
        out_shape=jax.ShapeDtypeStruct((B, _H0, 32), x.dtype),
        grid=(B // n,),
        in_specs=in_specs,
        out_specs=pl.BlockSpec((n, _H0, 32), lambda i: (i, 0, 0)),
        compiler_params=pltpu.CompilerParams(
            dimension_semantics=("parallel",),
            vmem_limit_bytes=64 * 1024 * 1024,
        ),
    )(x, mt, sel0, sel1, *flat)

    return out


# speedup vs baseline: 1.0426x; 1.0426x over previous
import functools
import math

import jax
import jax.numpy as jnp
import numpy as np
from jax import lax
from jax.experimental import pallas as pl
from jax.experimental.pallas import tpu as pltpu

_EPS = 1e-5
_N = 4
_H0 = 256
_H1 = 128
_K = 5

_BF = jnp.bfloat16


def _shiftr(xe, d, H):
    if d == 0:
        return xe
    z = jnp.zeros((abs(d), xe.shape[1]), xe.dtype)
    if d > 0:
        return jnp.concatenate([xe[d:, :], z], axis=0)
    return jnp.concatenate([z, xe[:H + d, :]], axis=0)


def _conv_same(x, w3, *, K, H, n):
    pad = K // 2
    xb = [x[e * H:(e + 1) * H, :].astype(_BF) for e in range(n)]
    acc = None
    for t in range(K):
        d = t - pad
        shf = jnp.concatenate([_shiftr(xe, d, H) for xe in xb], axis=0)
        dt = jnp.dot(shf, w3[t], preferred_element_type=jnp.float32)
        acc = dt if acc is None else acc + dt
    return acc


def _gn_mish(acc, sel, b, g, beta, gavg, *, H, n, post=None):
    cout = acc.shape[1]
    cc = jnp.concatenate([acc, jnp.square(acc)], axis=1)
    s = jnp.dot(sel, cc, preferred_element_type=jnp.float32)
    m_raw = s[:, :cout]
    q_raw = s[:, cout:]
    m = m_raw + b
    q = q_raw + 2.0 * b * m_raw + b * b
    gs = jnp.dot(jnp.concatenate([m, q], axis=0), gavg,
                 preferred_element_type=jnp.float32)
    gm = gs[:n, :]
    var = jnp.maximum(gs[n:, :] - gm * gm, 0.0)
    sc = g * lax.rsqrt(var + _EPS)
    sh = (b - gm) * sc + beta
    outs = []
    for e in range(n):
        y = acc[e * H:(e + 1) * H, :] * sc[e:e + 1, :] + sh[e:e + 1, :]
        t2 = jnp.square(1.0 + jnp.exp(jnp.minimum(y, 20.0)))
        o = y * (t2 - 1.0) / (t2 + 1.0)
        if post is not None:
            o = o + post[e:e + 1, :]
        outs.append(o)
    return jnp.concatenate(outs, axis=0)


def _rb(x, mtm, sel, w0, w1, tw, pvt, gavg, wr, *, K, H, n):
    pv = pvt[...]
    tb = jnp.dot(mtm, tw[...], preferred_element_type=jnp.float32) \
        + pv[7:8, :]
    a0 = _conv_same(x, w0, K=K, H=H, n=n)
    h = _gn_mish(a0, sel, pv[0:1, :], pv[1:2, :], pv[2:3, :], gavg,
                 H=H, n=n, post=tb)
    a1 = _conv_same(h, w1, K=K, H=H, n=n)
    y = _gn_mish(a1, sel, pv[3:4, :], pv[4:5, :], pv[5:6, :], gavg, H=H, n=n)
    if wr is None:
        res = x
    else:
        res = jnp.dot(x.astype(_BF), wr[...],
                      preferred_element_type=jnp.float32) + pv[6:7, :]
    return y + res


def _unet_kernel(x_ref, mt_ref, sel0_ref, sel1_ref,
                 a_w0, a_w1, a_tw, a_pv, a_wr,
                 b_w0, b_w1, b_tw, b_pv,
                 g256, g512,
                 dw3, dsel, ddb,
                 c_w0, c_w1, c_tw, c_pv, c_wr,
                 d_w0, d_w1, d_tw, d_pv,
                 e_w0, e_w1, e_tw, e_pv,
                 f_w0_, f_w1, f_tw, f_pv,
                 p_w0, p_w1, p_tw, p_pv, p_wr,
                 q_w0, q_w1, q_tw, q_pv,
                 uwe, uwo, upe, upo, uub,
                 z_w0, z_pv, z_wf, z_bf,
                 o_ref, *, n):
    H0, H1, K = _H0, _H1, _K
    ga = g256[...]
    gb = g512[...]
    sel0 = sel0_ref[...]
    sel1 = sel1_ref[...]

    x = jnp.concatenate([x_ref[e].astype(jnp.float32) for e in range(n)],
                        axis=0)
    mtm = jnp.concatenate([mt_ref[e] for e in range(n)],
                          axis=0).astype(_BF)

    x = _rb(x, mtm, sel0, a_w0, a_w1, a_tw, a_pv, ga, a_wr, K=K, H=H0, n=n)
    x = _rb(x, mtm, sel0, b_w0, b_w1, b_tw, b_pv, ga, None, K=K, H=H0, n=n)

    parts = []
    for e in range(n):
        xe = x[e * H0:(e + 1) * H0, :].astype(_BF)
        gath = jnp.dot(dsel[...], xe,
                       preferred_element_type=jnp.float32)
        gb16 = gath.astype(_BF)
        acc = None
        for t in range(3):
            dt = jnp.dot(gb16[t * H1:(t + 1) * H1, :], dw3[t],
                         preferred_element_type=jnp.float32)
            acc = dt if acc is None else acc + dt
        parts.append(acc)
    x = jnp.concatenate(parts, axis=0) + ddb[...]

    x = _rb(x, mtm, sel1, c_w0, c_w1, c_tw, c_pv, gb, c_wr, K=K, H=H1, n=n)
    x = _rb(x, mtm, sel1, d_w0, d_w1, d_tw, d_pv, gb, None, K=K, H=H1, n=n)
    skip = x

    x = _rb(x, mtm, sel1, e_w0, e_w1, e_tw, e_pv, gb, None, K=K, H=H1, n=n)
    x = _rb(x, mtm, sel1, f_w0_, f_w1, f_tw, f_pv, gb, None, K=K, H=H1, n=n)

    x = jnp.concatenate([x, skip], axis=1)
    x = _rb(x, mtm, sel1, p_w0, p_w1, p_tw, p_pv, ga, p_wr, K=K, H=H1, n=n)
    x = _rb(x, mtm, sel1, q_w0, q_w1, q_tw, q_pv, ga, None, K=K, H=H1, n=n)

    parts = []
    for e in range(n):
        xe = x[e * H1:(e + 1) * H1, :].astype(_BF)
        xm1 = _shiftr(xe, -1, H1)
        xp1 = _shiftr(xe, 1, H1)
        ev = jnp.dot(jnp.concatenate([xm1, xe], axis=1), uwe[...],
                     preferred_element_type=jnp.float32)
        od = jnp.dot(jnp.concatenate([xe, xp1], axis=1), uwo[...],
                     preferred_element_type=jnp.float32)
        parts.append(
            jnp.dot(upe[...], ev.astype(_BF),
                    preferred_element_type=jnp.float32)
            + jnp.dot(upo[...], od.astype(_BF),
                      preferred_element_type=jnp.float32))
    x = jnp.concatenate(parts, axis=0) + uub[...]

    pvz = z_pv[...]
    a = _conv_same(x, z_w0, K=K, H=H0, n=n)
    y = _gn_mish(a, sel0, pvz[0:1, :], pvz[1:2, :], pvz[2:3, :], ga,
                 H=H0, n=n)
    out = jnp.dot(y.astype(_BF), z_wf[...],
                  preferred_element_type=jnp.float32) + z_bf[...]
    for e in range(n):
        o_ref[e] = out[e * H0:(e + 1) * H0, :]


def _mish(v):
    return v * jnp.tanh(jax.nn.softplus(v))


def _sin_emb(t, dim):
    half = dim // 2
    freq = jnp.exp(jnp.arange(half, dtype=jnp.float32)
                   * (-math.log(10000.0) / (half - 1)))
    args = t.astype(jnp.float32)[:, None] * freq[None, :]
    return jnp.concatenate([jnp.sin(args), jnp.cos(args)], axis=-1)


def _cspec(shape):
    nd = len(shape)
    return pl.BlockSpec(shape, lambda i: (0,) * nd)


@functools.lru_cache(maxsize=None)
def _sel_matrices():
    s = np.zeros((3 * _H1, _H0), np.float32)
    for t in range(3):
        for j in range(_H1):
            m = 2 * j + t - 1
            if 0 <= m < _H0:
                s[t * _H1 + j, m] = 1.0
    pe = np.zeros((_H0, _H1), np.float32)
    po = np.zeros((_H0, _H1), np.float32)
    pe[2 * np.arange(_H1), np.arange(_H1)] = 1.0
    po[2 * np.arange(_H1) + 1, np.arange(_H1)] = 1.0
    s0 = np.zeros((_N, _N * _H0), np.float32)
    s1 = np.zeros((_N, _N * _H1), np.float32)
    for e in range(_N):
        s0[e, e * _H0:(e + 1) * _H0] = 1.0 / _H0
        s1[e, e * _H1:(e + 1) * _H1] = 1.0 / _H1
    return s, pe, po, s0, s1


def _tapw(w, cin):
    cout = w.shape[0]
    k = w.shape[1] // cin
    return jnp.transpose(w.reshape(cout, k, cin), (1, 2, 0)).astype(_BF)


def kernel(tm_w1, tm_b1, tm_w2, tm_b2, rm_w1, rm_b1, rm_w2, rm_b2, rm_w3,
           rm_b3, d0r1_w0, d0r1_w1, d0r1_tw, d0r1_pv, d0r1_gavg, d0r1_wr,
           d0r2_w0, d0r2_w1, d0r2_tw, d0r2_pv, d0r2_gavg, d0_dw, d0_db,
           d1r1_w0, d1r1_w1, d1r1_tw, d1r1_pv, d1r1_gavg, d1r1_wr,
           d1r2_w0, d1r2_w1, d1r2_tw, d1r2_pv, d1r2_gavg,
           m1_w0, m1_w1, m1_tw, m1_pv, m1_gavg,
           m2_w0, m2_w1, m2_tw, m2_pv, m2_gavg,
           u0r1_w0, u0r1_w1, u0r1_tw, u0r1_pv, u0r1_gavg, u0r1_wr,
           u0r2_w0, u0r2_w1, u0r2_tw, u0r2_pv, u0r2_gavg,
           u0_uw, u0_ub, f_w0, f_pv, f_gavg, f_wf, f_bf, x, time, returns):
    B = x.shape[0]
    n = _N
    dim = 256

    e = _sin_emb(time, dim)
    e = _mish(jnp.dot(e, tm_w1.T) + tm_b1)
    t = jnp.dot(e, tm_w2.T) + tm_b2
    r = _mish(jnp.dot(returns, rm_w1.T) + rm_b1)
    r = _mish(jnp.dot(r, rm_w2.T) + rm_b2)
    r = jnp.dot(r, rm_w3.T) + rm_b3
    mt = _mish(jnp.concatenate([t, r], axis=-1))
    mt = mt.reshape(B, 1, 512)

    bf = lambda a: a.astype(_BF)

    dw3 = bf(jnp.transpose(d0_dw, (2, 1, 0)))
    ddb = d0_db.reshape(1, -1)

    uwe = bf(jnp.concatenate([u0_uw[:, :, 0].T, u0_uw[:, :, 2].T], axis=0))
    uwo = bf(jnp.concatenate([u0_uw[:, :, 1].T, u0_uw[:, :, 3].T], axis=0))
    uub = u0_ub.reshape(1, -1)

    s_np, pe_np, po_np, s0_np, s1_np = _sel_matrices()
    dsel = jnp.asarray(s_np, _BF)
    upe = jnp.asarray(pe_np, _BF)
    upo = jnp.asarray(po_np, _BF)
    sel0 = jnp.asarray(s0_np)
    sel1 = jnp.asarray(s1_np)

    pvt = lambda p: p.T

    consts = [
        (_tapw(d0r1_w0, 32), _tapw(d0r1_w1, 256), bf(d0r1_tw.T),
         pvt(d0r1_pv), bf(d0r1_wr.T)),
        (_tapw(d0r2_w0, 256), _tapw(d0r2_w1, 256), bf(d0r2_tw.T),
         pvt(d0r2_pv)),
        (d0r1_gavg, d1r1_gavg),
        (dw3, dsel, ddb),
        (_tapw(d1r1_w0, 256), _tapw(d1r1_w1, 512), bf(d1r1_tw.T),
         pvt(d1r1_pv), bf(d1r1_wr.T)),
        (_tapw(d1r2_w0, 512), _tapw(d1r2_w1, 512), bf(d1r2_tw.T),
         pvt(d1r2_pv)),
        (_tapw(m1_w0, 512), _tapw(m1_w1, 512), bf(m1_tw.T), pvt(m1_pv)),
        (_tapw(m2_w0, 512), _tapw(m2_w1, 512), bf(m2_tw.T), pvt(m2_pv)),
        (_tapw(u0r1_w0, 1024), _tapw(u0r1_w1, 256), bf(u0r1_tw.T),
         pvt(u0r1_pv), bf(u0r1_wr.T)),
        (_tapw(u0r2_w0, 256), _tapw(u0r2_w1, 256), bf(u0r2_tw.T),
         pvt(u0r2_pv)),
        (uwe, uwo, upe, upo, uub),
        (_tapw(f_w0, 256), pvt(f_pv), bf(f_wf.T), f_bf.reshape(1, -1)),
    ]
    flat = [a for grp in consts for a in grp]

    in_specs = [
        pl.BlockSpec((n, _H0, 32), lambda i: (i, 0, 0)),
        pl.BlockSpec((n, 1, 512), lambda i: (i, 0, 0)),
        _cspec(sel0.shape),
        _cspec(sel1.shape),
    ] + [_cspec(a.shape) for a in flat]

    out = pl.pallas_call(
        functools.partial(_unet_kernel, n=n),
        out_shape=jax.ShapeDtypeStruct((B, _H0, 32), x.dtype),
        grid=(B // n,),
        in_specs=in_specs,
        out_specs=pl.BlockSpec((n, _H0, 32), lambda i: (i, 0, 0)),
        compiler_params=pltpu.CompilerParams(
            dimension_semantics=("parallel",),
            vmem_limit_bytes=64 * 1024 * 1024,
        ),
    )(x, mt, sel0, sel1, *flat)

    return out
